# baseline (speedup 1.0000x reference)
import functools

import jax
import jax.numpy as jnp
from jax import lax
from jax.experimental import pallas as pl
from jax.experimental.pallas import tpu as pltpu
from jax.experimental.pallas import tpu_sc as plsc

NC = 2
NS = 16
NW = NC * NS
LANES = 16
CHUNK = 128
NBUF = 8


def _cdiv(a, b):
    return (a + b - 1) // b


def _deg_kernel(nt, epw, nw):
    mesh = plsc.VectorSubcoreMesh(core_axis_name="c", subcore_axis_name="s")

    @functools.partial(
        pl.kernel,
        out_type=jax.ShapeDtypeStruct((nw, nt), jnp.float32),
        mesh=mesh,
        scratch_types=[
            pltpu.VMEM((epw,), jnp.int32),
            pltpu.VMEM((nt,), jnp.float32),
        ],
        compiler_params=pltpu.CompilerParams(needs_layout_passes=False),
    )
    def degk(dflat_hbm, hist_hbm, idx_v, hist_v):
        cid = lax.axis_index("c")
        sid = lax.axis_index("s")
        wid = sid * NC + cid
        pltpu.sync_copy(dflat_hbm.at[pl.ds(wid * epw, epw)], idx_v)

        def zbody(i, carry):
            hist_v[pl.ds(i * LANES, LANES)] = jnp.zeros((LANES,), jnp.float32)
            return carry

        lax.fori_loop(0, nt // LANES, zbody, 0)
        ones = jnp.ones((LANES,), jnp.float32)

        def ebody(w, carry):
            idx = idx_v[pl.ds(w * LANES, LANES)]
            plsc.addupdate_scatter(hist_v, [idx], ones)
            return carry

        lax.fori_loop(0, epw // LANES, ebody, 0)
        pltpu.sync_copy(hist_v, hist_hbm.at[wid])

    return degk


def _main_kernel(n16, fh, nch):
    mesh = plsc.VectorSubcoreMesh(core_axis_name="c", subcore_axis_name="s")
    rp = n16 // NS
    CR = rp // 4

    @functools.partial(
        pl.kernel,
        out_type=(
            jax.ShapeDtypeStruct((n16, 2 * fh), jnp.float32),
            jax.ShapeDtypeStruct((NC * n16, fh), jnp.float32),
        ),
        mesh=mesh,
        scratch_types=[
            pltpu.VMEM((nch, CHUNK), jnp.int32),
            pltpu.VMEM((nch, CHUNK), jnp.int32),
            pltpu.VMEM((NBUF, CHUNK, fh), jnp.float32),
            pltpu.VMEM((CR, fh), jnp.float32),
            pltpu.VMEM((CR, fh), jnp.float32),
            pltpu.VMEM((rp,), jnp.float32),
            pltpu.VMEM((rp,), jnp.float32),
            pltpu.VMEM((2 * fh,), jnp.float32),
            pltpu.VMEM_SHARED((n16, fh), jnp.float32),
            pltpu.SemaphoreType.DMA((NBUF,)),
            pltpu.SemaphoreType.DMA((NBUF,)),
        ],
        compiler_params=pltpu.CompilerParams(
            needs_layout_passes=False, use_tc_tiling_on_sc=False),
    )
    def maink(p0_hbm, s_hbm, d_hbm, z_hbm, dinv_hbm, dinvs_hbm, b_hbm,
              out_hbm, p1_hbm,
              idx_s, idx_d, rows, cbA, cbB, dv, dv2, bv, sp_t,
              gsem, ssem):
        cid = lax.axis_index("c")
        sid = lax.axis_index("s")
        r0 = sid * rp

        pltpu.sync_copy(z_hbm, sp_t.at[pl.ds(r0, rp)])
        pltpu.sync_copy(s_hbm.at[cid].at[sid], idx_s)
        pltpu.sync_copy(d_hbm.at[sid], idx_d)
        pltpu.sync_copy(dinv_hbm.at[pl.ds(r0, rp)], dv)
        pltpu.sync_copy(dinvs_hbm.at[pl.ds(r0, rp)], dv2)
        pltpu.sync_copy(b_hbm, bv)

        def edge_loop(table_hbm):
            def body(i, carry):
                descs = []
                for bb in range(NBUF):
                    j = i * NBUF + bb
                    descs.append(pltpu.async_copy(
                        table_hbm.at[idx_s.at[j]], rows.at[bb], gsem.at[bb]))
                sdescs = []
                for bb in range(NBUF):
                    j = i * NBUF + bb
                    descs[bb].wait()
                    sdescs.append(pltpu.async_copy(
                        rows.at[bb], sp_t.at[idx_d.at[j]], ssem.at[bb],
                        add=True))
                for bb in range(NBUF):
                    sdescs[bb].wait()
                return carry
            lax.fori_loop(0, nch // NBUF, body, 0)

        def rescale(prev_hbm, scale_ref, final):
            if final:
                bq0 = bv[pl.ds(cid * fh, 16)]
                bq1 = bv[pl.ds(cid * fh + 16, 16)]
            for h in range(4):
                base = r0 + h * CR
                pltpu.sync_copy(sp_t.at[pl.ds(base, CR)], cbA)
                pltpu.sync_copy(prev_hbm.at[pl.ds(cid * n16 + base, CR)], cbB)

                def rowbody(r, carry):
                    sc = plsc.load_gather(
                        scale_ref, [jnp.full((LANES,), h * CR + r, jnp.int32)])
                    for q in range(fh // LANES):
                        sl = pl.ds(q * LANES, LANES)
                        v = (cbA[r, sl] + cbB[r, sl]) * sc
                        if final:
                            v = v + (bq0 if q == 0 else bq1)
                        cbA[r, sl] = v
                    return carry
                lax.fori_loop(0, CR, rowbody, 0)
                if final:
                    pltpu.sync_copy(
                        cbA, out_hbm.at[pl.ds(base, CR), pl.ds(cid * fh, fh)])
                else:
                    pltpu.sync_copy(
                        cbA, p1_hbm.at[pl.ds(cid * n16 + base, CR)])
                    pltpu.sync_copy(z_hbm.at[pl.ds(h * CR, CR)],
                                    sp_t.at[pl.ds(base, CR)])

        plsc.subcore_barrier()
        edge_loop(p0_hbm)
        plsc.subcore_barrier()
        rescale(p0_hbm, dv, final=False)
        plsc.subcore_barrier()
        edge_loop(p1_hbm)
        plsc.subcore_barrier()
        rescale(p1_hbm, dv2, final=True)

    return maink


def _tc_prep_body(x_ref, w_ref, hist_ref, p0_ref, dinv_ref, dinvs_ref,
                  *, n, n16, fh):
    deg = jnp.sum(hist_ref[...], axis=0) + 1.0
    dinv_ref[...] = 1.0 / deg
    dinvs_ref[...] = lax.rsqrt(deg)
    xw = lax.dot_general(x_ref[...], w_ref[...],
                         (((1,), (1,)), ((), ())),
                         preferred_element_type=jnp.float32)
    xws = xw * dinvs_ref[pl.ds(0, n)][:, None]
    zpad = jnp.zeros((n16 - n, fh), jnp.float32)
    p0_ref[pl.ds(0, n16)] = jnp.concatenate([xws[:, :fh], zpad], axis=0)
    p0_ref[pl.ds(n16, n16)] = jnp.concatenate([xws[:, fh:], zpad], axis=0)


def kernel(x, edge_index, W, b):
    n, f_in = x.shape
    fo = W.shape[0]
    fh = fo // 2
    e = edge_index.shape[1]

    n16 = NS * 8 * _cdiv(n + 1, NS * 8)

    src = edge_index[0]
    dst = edge_index[1]

    epw1 = CHUNK * _cdiv(_cdiv(e, NW), CHUNK)
    dst_p1 = jnp.concatenate(
        [dst, jnp.full((NW * epw1 - e,), n, jnp.int32)])

    nch = NBUF * _cdiv(_cdiv(e, NS), CHUNK * NBUF)
    epw2 = nch * CHUNK
    src_p2 = jnp.concatenate([src, jnp.zeros((NS * epw2 - e,), jnp.int32)])
    dst_p2 = jnp.concatenate([dst, jnp.full((NS * epw2 - e,), n, jnp.int32)])
    src3 = src_p2.reshape(NS, nch, CHUNK)
    dst3 = dst_p2.reshape(NS, nch, CHUNK)
    src4 = jnp.stack([src3, src3 + n16])

    rp = n16 // NS
    zrows = jnp.zeros((rp, fh), jnp.float32)

    hist = _deg_kernel(n16, epw1, NW)(dst_p1)

    tc_prep = pl.pallas_call(
        functools.partial(_tc_prep_body, n=n, n16=n16, fh=fh),
        out_shape=(
            jax.ShapeDtypeStruct((NC * n16, fh), jnp.float32),
            jax.ShapeDtypeStruct((n16,), jnp.float32),
            jax.ShapeDtypeStruct((n16,), jnp.float32),
        ),
    )
    p0, dinv, dinvs = tc_prep(x, W, hist)

    out, _ = _main_kernel(n16, fh, nch)(
        p0, src4, dst3, zrows, dinv, dinvs, b)
    return out[:n]


# speedup vs baseline: 1.6129x; 1.6129x over previous
import functools

import jax
import jax.numpy as jnp
from jax import lax
from jax.experimental import pallas as pl
from jax.experimental.pallas import tpu as pltpu
from jax.experimental.pallas import tpu_sc as plsc

NC = 2
NS = 16
NW = NC * NS
LANES = 16
CHUNK = 128
NBUF = 8


def _cdiv(a, b):
    return (a + b - 1) // b


def _deg_kernel(nt, epw, nw):
    mesh = plsc.VectorSubcoreMesh(core_axis_name="c", subcore_axis_name="s")

    @functools.partial(
        pl.kernel,
        out_type=jax.ShapeDtypeStruct((nw, nt), jnp.float32),
        mesh=mesh,
        scratch_types=[
            pltpu.VMEM((epw,), jnp.int32),
            pltpu.VMEM((nt,), jnp.float32),
        ],
        compiler_params=pltpu.CompilerParams(needs_layout_passes=False),
    )
    def degk(dflat_hbm, hist_hbm, idx_v, hist_v):
        cid = lax.axis_index("c")
        sid = lax.axis_index("s")
        wid = sid * NC + cid
        pltpu.sync_copy(dflat_hbm.at[pl.ds(wid * epw, epw)], idx_v)

        def zbody(i, carry):
            hist_v[pl.ds(i * LANES, LANES)] = jnp.zeros((LANES,), jnp.float32)
            return carry

        lax.fori_loop(0, nt // LANES, zbody, 0)
        ones = jnp.ones((LANES,), jnp.float32)

        def ebody(w, carry):
            idx = idx_v[pl.ds(w * LANES, LANES)]
            plsc.addupdate_scatter(hist_v, [idx], ones)
            return carry

        lax.fori_loop(0, epw // LANES, ebody, 0)
        pltpu.sync_copy(hist_v, hist_hbm.at[wid])

    return degk


def _main_kernel(n16, fh, nch):
    mesh = plsc.VectorSubcoreMesh(core_axis_name="c", subcore_axis_name="s")
    rp = n16 // NS
    CR = rp // 4

    @functools.partial(
        pl.kernel,
        out_type=jax.ShapeDtypeStruct((n16, 2 * fh), jnp.float32),
        mesh=mesh,
        scratch_types=[
            pltpu.VMEM((nch, CHUNK), jnp.int32),
            pltpu.VMEM((nch, CHUNK), jnp.int32),
            pltpu.VMEM((NBUF, CHUNK, fh), jnp.float32),
            pltpu.VMEM((CR, fh), jnp.float32),
            pltpu.VMEM((CR, fh), jnp.float32),
            pltpu.VMEM((rp,), jnp.float32),
            pltpu.VMEM((rp,), jnp.float32),
            pltpu.VMEM((2 * fh,), jnp.float32),
            pltpu.VMEM_SHARED((n16, fh), jnp.float32),
            pltpu.VMEM_SHARED((n16, fh), jnp.float32),
            pltpu.SemaphoreType.DMA((NBUF,)),
            pltpu.SemaphoreType.DMA((NBUF,)),
        ],
        compiler_params=pltpu.CompilerParams(
            needs_layout_passes=False, use_tc_tiling_on_sc=False),
    )
    def maink(p0_hbm, s_hbm, d_hbm, z_hbm, dinv_hbm, dinvs_hbm, b_hbm, out_hbm,
              idx_s, idx_d, rows, cbA, cbB, dv, dv2, bv, sp_p, sp_t,
              gsem, ssem):
        cid = lax.axis_index("c")
        sid = lax.axis_index("s")
        r0 = sid * rp

        pltpu.sync_copy(p0_hbm.at[cid].at[pl.ds(r0, rp)], sp_p.at[pl.ds(r0, rp)])
        pltpu.sync_copy(z_hbm, sp_t.at[pl.ds(r0, rp)])
        pltpu.sync_copy(s_hbm.at[sid], idx_s)
        pltpu.sync_copy(d_hbm.at[sid], idx_d)
        pltpu.sync_copy(dinv_hbm.at[pl.ds(r0, rp)], dv)
        pltpu.sync_copy(dinvs_hbm.at[pl.ds(r0, rp)], dv2)
        pltpu.sync_copy(b_hbm, bv)

        def edge_loop():
            nb = nch // NBUF

            def body(i, carry):
                descs = []
                for bb in range(NBUF):
                    j = i * NBUF + bb

                    @pl.when(i > 0)
                    def _():
                        pltpu.make_async_copy(
                            rows.at[bb], sp_t.at[idx_d.at[j]],
                            ssem.at[bb]).wait()
                    descs.append(pltpu.async_copy(
                        sp_p.at[idx_s.at[j]], rows.at[bb], gsem.at[bb]))
                for bb in range(NBUF):
                    j = i * NBUF + bb
                    descs[bb].wait()
                    pltpu.async_copy(
                        rows.at[bb], sp_t.at[idx_d.at[j]], ssem.at[bb],
                        add=True)
                return carry
            lax.fori_loop(0, nb, body, 0)
            for bb in range(NBUF):
                j = (nb - 1) * NBUF + bb
                pltpu.make_async_copy(
                    rows.at[bb], sp_t.at[idx_d.at[j]], ssem.at[bb]).wait()

        def rescale(scale_ref, final):
            if final:
                bq0 = bv[pl.ds(cid * fh, 16)]
                bq1 = bv[pl.ds(cid * fh + 16, 16)]
            for h in range(rp // CR):
                base = r0 + h * CR
                pltpu.sync_copy(sp_t.at[pl.ds(base, CR)], cbA)
                pltpu.sync_copy(sp_p.at[pl.ds(base, CR)], cbB)

                def rowbody(r, carry):
                    sc = plsc.load_gather(
                        scale_ref, [jnp.full((LANES,), h * CR + r, jnp.int32)])
                    for q in range(fh // LANES):
                        sl = pl.ds(q * LANES, LANES)
                        v = (cbA[r, sl] + cbB[r, sl]) * sc
                        if final:
                            v = v + (bq0 if q == 0 else bq1)
                        cbA[r, sl] = v
                    return carry
                lax.fori_loop(0, CR, rowbody, 0)
                if final:
                    pltpu.sync_copy(
                        cbA, out_hbm.at[pl.ds(base, CR), pl.ds(cid * fh, fh)])
                else:
                    pltpu.sync_copy(cbA, sp_p.at[pl.ds(base, CR)])
                    pltpu.sync_copy(z_hbm.at[pl.ds(h * CR, CR)],
                                    sp_t.at[pl.ds(base, CR)])

        plsc.subcore_barrier()
        edge_loop()
        plsc.subcore_barrier()
        rescale(dv, final=False)
        plsc.subcore_barrier()
        edge_loop()
        plsc.subcore_barrier()
        rescale(dv2, final=True)

    return maink


def _tc_prep_body(x_ref, w_ref, hist_ref, p0_ref, dinv_ref, dinvs_ref,
                  *, n, n16, fh):
    deg = jnp.sum(hist_ref[...], axis=0) + 1.0
    dinv_ref[...] = 1.0 / deg
    dinvs_ref[...] = lax.rsqrt(deg)
    xw = lax.dot_general(x_ref[...], w_ref[...],
                         (((1,), (1,)), ((), ())),
                         preferred_element_type=jnp.float32)
    xws = xw * dinvs_ref[pl.ds(0, n)][:, None]
    zpad = jnp.zeros((n16 - n, fh), jnp.float32)
    p0_ref[0] = jnp.concatenate([xws[:, :fh], zpad], axis=0)
    p0_ref[1] = jnp.concatenate([xws[:, fh:], zpad], axis=0)


def kernel(x, edge_index, W, b):
    n, f_in = x.shape
    fo = W.shape[0]
    fh = fo // 2
    e = edge_index.shape[1]

    n16 = NS * 8 * _cdiv(n + 1, NS * 8)

    src = edge_index[0]
    dst = edge_index[1]

    epw1 = CHUNK * _cdiv(_cdiv(e, NW), CHUNK)
    dst_p1 = jnp.concatenate(
        [dst, jnp.full((NW * epw1 - e,), n, jnp.int32)])

    nch = NBUF * _cdiv(_cdiv(e, NS), CHUNK * NBUF)
    epw2 = nch * CHUNK
    src_p2 = jnp.concatenate([src, jnp.zeros((NS * epw2 - e,), jnp.int32)])
    dst_p2 = jnp.concatenate([dst, jnp.full((NS * epw2 - e,), n, jnp.int32)])
    src3 = src_p2.reshape(NS, nch, CHUNK)
    dst3 = dst_p2.reshape(NS, nch, CHUNK)

    rp = n16 // NS
    zrows = jnp.zeros((rp, fh), jnp.float32)

    hist = _deg_kernel(n16, epw1, NW)(dst_p1)

    tc_prep = pl.pallas_call(
        functools.partial(_tc_prep_body, n=n, n16=n16, fh=fh),
        out_shape=(
            jax.ShapeDtypeStruct((2, n16, fh), jnp.float32),
            jax.ShapeDtypeStruct((n16,), jnp.float32),
            jax.ShapeDtypeStruct((n16,), jnp.float32),
        ),
    )
    p0, dinv, dinvs = tc_prep(x, W, hist)

    out = _main_kernel(n16, fh, nch)(p0, src3, dst3, zrows, dinv, dinvs, b)
    return out[:n]


# speedup vs baseline: 1.6446x; 1.0197x over previous
import functools

import jax
import jax.numpy as jnp
from jax import lax
from jax.experimental import pallas as pl
from jax.experimental.pallas import tpu as pltpu
from jax.experimental.pallas import tpu_sc as plsc

NC = 2
NS = 16
NW = NC * NS
LANES = 16
CHUNK = 128
NBUF = 8


def _cdiv(a, b):
    return (a + b - 1) // b


def _deg_kernel(nt, epw, nw):
    mesh = plsc.VectorSubcoreMesh(core_axis_name="c", subcore_axis_name="s")

    @functools.partial(
        pl.kernel,
        out_type=jax.ShapeDtypeStruct((nw, nt), jnp.float32),
        mesh=mesh,
        scratch_types=[
            pltpu.VMEM((epw,), jnp.int32),
            pltpu.VMEM((nt,), jnp.float32),
        ],
        compiler_params=pltpu.CompilerParams(needs_layout_passes=False),
    )
    def degk(dflat_hbm, hist_hbm, idx_v, hist_v):
        cid = lax.axis_index("c")
        sid = lax.axis_index("s")
        wid = sid * NC + cid
        pltpu.sync_copy(dflat_hbm.at[pl.ds(wid * epw, epw)], idx_v)

        def zbody(i, carry):
            hist_v[pl.ds(i * LANES, LANES)] = jnp.zeros((LANES,), jnp.float32)
            return carry

        lax.fori_loop(0, nt // LANES, zbody, 0)
        ones = jnp.ones((LANES,), jnp.float32)

        def ebody(w, carry):
            idx = idx_v[pl.ds(w * LANES, LANES)]
            plsc.addupdate_scatter(hist_v, [idx], ones)
            return carry

        lax.fori_loop(0, epw // LANES, ebody, 0)
        pltpu.sync_copy(hist_v, hist_hbm.at[wid])

    return degk


def _main_kernel(n16, fh, nch):
    mesh = plsc.VectorSubcoreMesh(core_axis_name="c", subcore_axis_name="s")
    rp = n16 // NS
    CR = rp // 4

    @functools.partial(
        pl.kernel,
        out_type=jax.ShapeDtypeStruct((n16, 2 * fh), jnp.float32),
        mesh=mesh,
        scratch_types=[
            pltpu.VMEM((nch, CHUNK), jnp.int32),
            pltpu.VMEM((nch, CHUNK), jnp.int32),
            pltpu.VMEM((NBUF, CHUNK, fh), jnp.float32),
            pltpu.VMEM((CR, fh), jnp.float32),
            pltpu.VMEM((CR, fh), jnp.float32),
            pltpu.VMEM((rp,), jnp.float32),
            pltpu.VMEM((rp,), jnp.float32),
            pltpu.VMEM((2 * fh,), jnp.float32),
            pltpu.VMEM_SHARED((n16, fh), jnp.float32),
            pltpu.VMEM_SHARED((n16, fh), jnp.float32),
            pltpu.SemaphoreType.DMA((NBUF,)),
            pltpu.SemaphoreType.DMA((NBUF,)),
        ],
        compiler_params=pltpu.CompilerParams(
            needs_layout_passes=False, use_tc_tiling_on_sc=False),
    )
    def maink(p0_hbm, s_hbm, d_hbm, z_hbm, dinv_hbm, dinvs_hbm, b_hbm, out_hbm,
              idx_s, idx_d, rows, cbA, cbB, dv, dv2, bv, sp_p, sp_t,
              gsem, ssem):
        cid = lax.axis_index("c")
        sid = lax.axis_index("s")
        r0 = sid * rp

        pltpu.sync_copy(p0_hbm.at[cid].at[pl.ds(r0, rp)], sp_p.at[pl.ds(r0, rp)])
        pltpu.sync_copy(z_hbm, sp_t.at[pl.ds(r0, rp)])
        pltpu.sync_copy(s_hbm.at[sid], idx_s)
        pltpu.sync_copy(d_hbm.at[sid], idx_d)
        pltpu.sync_copy(dinv_hbm.at[pl.ds(r0, rp)], dv)
        pltpu.sync_copy(dinvs_hbm.at[pl.ds(r0, rp)], dv2)
        pltpu.sync_copy(b_hbm, bv)

        def edge_loop():
            def body(i, carry):
                descs = []
                for bb in range(NBUF):
                    j = i * NBUF + bb
                    descs.append(pltpu.async_copy(
                        sp_p.at[idx_s.at[j]], rows.at[bb], gsem.at[bb]))
                sdescs = []
                for bb in range(NBUF):
                    j = i * NBUF + bb
                    descs[bb].wait()
                    sdescs.append(pltpu.async_copy(
                        rows.at[bb], sp_t.at[idx_d.at[j]], ssem.at[bb],
                        add=True))
                for bb in range(NBUF):
                    sdescs[bb].wait()
                return carry
            lax.fori_loop(0, nch // NBUF, body, 0)

        def rescale(scale_ref, final):
            if final:
                bq0 = bv[pl.ds(cid * fh, 16)]
                bq1 = bv[pl.ds(cid * fh + 16, 16)]
            for h in range(rp // CR):
                base = r0 + h * CR
                pltpu.sync_copy(sp_t.at[pl.ds(base, CR)], cbA)
                pltpu.sync_copy(sp_p.at[pl.ds(base, CR)], cbB)

                def rowbody(r, carry):
                    sc = plsc.load_gather(
                        scale_ref, [jnp.full((LANES,), h * CR + r, jnp.int32)])
                    for q in range(fh // LANES):
                        sl = pl.ds(q * LANES, LANES)
                        v = (cbA[r, sl] + cbB[r, sl]) * sc
                        if final:
                            v = v + (bq0 if q == 0 else bq1)
                        cbA[r, sl] = v
                    return carry
                lax.fori_loop(0, CR, rowbody, 0)
                if final:
                    pltpu.sync_copy(
                        cbA, out_hbm.at[pl.ds(base, CR), pl.ds(cid * fh, fh)])
                else:
                    pltpu.sync_copy(cbA, sp_p.at[pl.ds(base, CR)])
                    pltpu.sync_copy(z_hbm.at[pl.ds(h * CR, CR)],
                                    sp_t.at[pl.ds(base, CR)])

        plsc.subcore_barrier()
        edge_loop()
        plsc.subcore_barrier()
        rescale(dv, final=False)
        plsc.subcore_barrier()
        edge_loop()
        plsc.subcore_barrier()
        rescale(dv2, final=True)

    return maink


def _tc_mm_body(x_ref, w_ref, xw_ref, *, n, n16, fh):
    xw = lax.dot_general(x_ref[...], w_ref[...],
                         (((1,), (1,)), ((), ())),
                         preferred_element_type=jnp.float32)
    zpad = jnp.zeros((n16 - n, fh), jnp.float32)
    xw_ref[0] = jnp.concatenate([xw[:, :fh], zpad], axis=0)
    xw_ref[1] = jnp.concatenate([xw[:, fh:], zpad], axis=0)


def _tc_scale_body(xw_ref, hist_ref, p0_ref, dinv_ref, dinvs_ref, *, n16):
    deg = jnp.sum(hist_ref[...], axis=0) + 1.0
    dinvs = lax.rsqrt(deg)
    dinv_ref[...] = 1.0 / deg
    dinvs_ref[...] = dinvs
    p0_ref[0] = xw_ref[0] * dinvs[:, None]
    p0_ref[1] = xw_ref[1] * dinvs[:, None]


def kernel(x, edge_index, W, b):
    n, f_in = x.shape
    fo = W.shape[0]
    fh = fo // 2
    e = edge_index.shape[1]

    n16 = NS * 8 * _cdiv(n + 1, NS * 8)

    src = edge_index[0]
    dst = edge_index[1]

    epw1 = CHUNK * _cdiv(_cdiv(e, NW), CHUNK)
    dst_p1 = jnp.concatenate(
        [dst, jnp.full((NW * epw1 - e,), n, jnp.int32)])

    nch = NBUF * _cdiv(_cdiv(e, NS), CHUNK * NBUF)
    epw2 = nch * CHUNK
    src_p2 = jnp.concatenate([src, jnp.zeros((NS * epw2 - e,), jnp.int32)])
    dst_p2 = jnp.concatenate([dst, jnp.full((NS * epw2 - e,), n, jnp.int32)])
    src3 = src_p2.reshape(NS, nch, CHUNK)
    dst3 = dst_p2.reshape(NS, nch, CHUNK)

    rp = n16 // NS
    zrows = jnp.zeros((rp, fh), jnp.float32)

    hist = _deg_kernel(n16, epw1, NW)(dst_p1)

    xw = pl.pallas_call(
        functools.partial(_tc_mm_body, n=n, n16=n16, fh=fh),
        out_shape=jax.ShapeDtypeStruct((2, n16, fh), jnp.float32),
    )(x, W)

    p0, dinv, dinvs = pl.pallas_call(
        functools.partial(_tc_scale_body, n16=n16),
        out_shape=(
            jax.ShapeDtypeStruct((2, n16, fh), jnp.float32),
            jax.ShapeDtypeStruct((n16,), jnp.float32),
            jax.ShapeDtypeStruct((n16,), jnp.float32),
        ),
    )(xw, hist)

    out = _main_kernel(n16, fh, nch)(p0, src3, dst3, zrows, dinv, dinvs, b)
    return out[:n]
